# baseline (speedup 1.0000x reference)
import functools

import jax
import jax.numpy as jnp
from jax import lax
from jax.experimental import pallas as pl
from jax.experimental.pallas import tpu as pltpu
from jax.experimental.pallas import tpu_sc as plsc

N_CORES = 2
N_SUB = 16
N_WORKERS = N_CORES * N_SUB
EDGE_WIN = 128

def _dot(a, b, precision=None):
    return jax.lax.dot_general(
        a, b, (((1,), (0,)), ((), ())),
        precision=precision, preferred_element_type=jnp.float32)


def _seed_zero(acc_sh, zblk_hbm, s, n, rows_per_sub, tail_rows):
    row0 = s * rows_per_sub
    zr = zblk_hbm.shape[0]
    pltpu.sync_copy(zblk_hbm, acc_sh.at[pl.ds(row0, zr)])
    k = zr
    while k < rows_per_sub:
        m = min(k, rows_per_sub - k)
        pltpu.sync_copy(acc_sh.at[pl.ds(row0, m)],
                        acc_sh.at[pl.ds(row0 + k, m)])
        k += m
    if tail_rows:
        @pl.when(s == 0)
        def _():
            pltpu.sync_copy(acc_sh.at[pl.ds(row0, tail_rows)],
                            acc_sh.at[pl.ds(n - tail_rows, tail_rows)])


def _dump_acc(acc_sh, out_view, s, n, rows_per_sub, tail_rows):
    row0 = s * rows_per_sub
    pltpu.sync_copy(acc_sh.at[pl.ds(row0, rows_per_sub)],
                    out_view.at[pl.ds(row0, rows_per_sub)])
    if tail_rows:
        @pl.when(s == 0)
        def _():
            pltpu.sync_copy(acc_sh.at[pl.ds(n - tail_rows, tail_rows)],
                            out_view.at[pl.ds(n - tail_rows, tail_rows)])


def _edge_loop(table_ref, acc_sh, src_hbm, dst_hbm, ebase0, src_v, dst_v,
               rows_v, st_v, dt_v, sems, n_win, tail, before_loop=None):
    def src_copy(w, b):
        return pltpu.make_async_copy(
            src_hbm.at[pl.ds(ebase0 + w * EDGE_WIN, EDGE_WIN)],
            src_v.at[b], sems.at[0, b])

    def dst_copy(w, b):
        return pltpu.make_async_copy(
            dst_hbm.at[pl.ds(ebase0 + w * EDGE_WIN, EDGE_WIN)],
            dst_v.at[b], sems.at[1, b])

    def gather_copy(w, b):
        return pltpu.make_async_copy(
            table_ref.at[src_v.at[b]], rows_v.at[b], sems.at[2, b])

    def scatter_copy(b):
        return pltpu.make_async_copy(
            rows_v.at[b], acc_sh.at[dst_v.at[b]], sems.at[3, b])

    def scatter_start(b):
        pltpu.async_copy(rows_v.at[b], acc_sh.at[dst_v.at[b]],
                         sems.at[3, b], add=True)

    for j in range(3):
        src_copy(j, j).start()
    for j in range(2):
        dst_copy(j, j).start()
    for j in range(2):
        src_copy(j, j).wait()
        gather_copy(j, j).start()

    if before_loop is not None:
        before_loop()

    @pl.loop(0, n_win // 3)
    def _(i):
        w0 = 3 * i
        for j in range(3):
            w = w0 + j
            b = j
            b2 = (j + 2) % 3
            gather_copy(w, b).wait()

            @pl.when(w + 3 < n_win)
            def _():
                src_copy(w + 3, b).start()

            @pl.when(w >= 1)
            def _():
                scatter_copy(b2).wait()

            @pl.when(w + 2 < n_win)
            def _():
                dst_copy(w + 2, b2).start()
                src_copy(w + 2, b2).wait()
                gather_copy(w + 2, b2).start()

            dst_copy(w, b).wait()
            scatter_start(b)

    scatter_copy((n_win - 1) % 3).wait()

    if tail:
        base = ebase0 + n_win * EDGE_WIN
        st = pltpu.make_async_copy(
            src_hbm.at[pl.ds(base, tail)], st_v, sems.at[0, 0])
        dt = pltpu.make_async_copy(
            dst_hbm.at[pl.ds(base, tail)], dt_v, sems.at[1, 0])
        st.start()
        dt.start()
        st.wait()
        gt = pltpu.make_async_copy(
            table_ref.at[st_v], rows_v.at[0, pl.ds(0, tail)], sems.at[2, 0])
        gt.start()
        gt.wait()
        dt.wait()
        pltpu.sync_copy(rows_v.at[0, pl.ds(0, tail)], acc_sh.at[dt_v],
                        add=True)


def _sc_scratch(n, d, tail):
    return [
        pltpu.VMEM((3, EDGE_WIN), jnp.int32),
        pltpu.VMEM((3, EDGE_WIN), jnp.int32),
        pltpu.VMEM((3, EDGE_WIN, d), jnp.float32),
        pltpu.VMEM((max(tail, 8),), jnp.int32),
        pltpu.VMEM((max(tail, 8),), jnp.int32),
        pltpu.VMEM_SHARED((n, d), jnp.float32),
        pltpu.SemaphoreType.DMA((4, 3)),
    ]


def _sc_seed_plus_segment_sum(table, src, dst):
    n, d = table.shape
    e = src.shape[0]
    per_w = e // N_WORKERS
    n_win = per_w // EDGE_WIN // 3 * 3
    tail = per_w - n_win * EDGE_WIN
    rows_per_sub = (n // N_SUB) // 8 * 8
    tail_rows = n - rows_per_sub * N_SUB
    zblk = jnp.zeros((64, d), jnp.float32)

    mesh = plsc.VectorSubcoreMesh(core_axis_name="c", subcore_axis_name="s")

    @functools.partial(
        pl.kernel,
        out_type=jax.ShapeDtypeStruct((N_CORES, n, d), jnp.float32),
        mesh=mesh,
        scratch_types=_sc_scratch(n, d, tail),
    )
    def k(table_hbm, src_hbm, dst_hbm, zblk_hbm, out_hbm,
          src_v, dst_v, rows_v, st_v, dt_v, acc_sh, sems):
        c = lax.axis_index("c")
        s = lax.axis_index("s")
        wid = c * N_SUB + s
        base0 = wid * per_w

        def seed_and_sync():
            _seed_zero(acc_sh, zblk_hbm, s, n, rows_per_sub, tail_rows)
            plsc.subcore_barrier()

        _edge_loop(table_hbm, acc_sh, src_hbm, dst_hbm, base0, src_v, dst_v,
                   rows_v, st_v, dt_v, sems, n_win, tail,
                   before_loop=seed_and_sync)
        plsc.subcore_barrier()
        _dump_acc(acc_sh, out_hbm.at[c], s, n, rows_per_sub, tail_rows)

    return k(table, src, dst, zblk)


def _sc_seed_plus_segment_sum_halves(tables, src, dst):
    _, n, d = tables.shape
    e = src.shape[0]
    per_s = e // N_SUB
    n_win = per_s // EDGE_WIN // 3 * 3
    tail = per_s - n_win * EDGE_WIN
    rows_per_sub = (n // N_SUB) // 8 * 8
    tail_rows = n - rows_per_sub * N_SUB
    zblk = jnp.zeros((64, d), jnp.float32)

    mesh = plsc.VectorSubcoreMesh(core_axis_name="c", subcore_axis_name="s")

    @functools.partial(
        pl.kernel,
        out_type=jax.ShapeDtypeStruct((N_CORES, n, d), jnp.float32),
        mesh=mesh,
        scratch_types=_sc_scratch(n, d, tail),
    )
    def k(tables_hbm, src_hbm, dst_hbm, zblk_hbm, out_hbm,
          src_v, dst_v, rows_v, st_v, dt_v, acc_sh, sems):
        c = lax.axis_index("c")
        s = lax.axis_index("s")
        base0 = s * per_s

        def run(table_ref):
            def seed_and_sync():
                _seed_zero(acc_sh, zblk_hbm, s, n, rows_per_sub, tail_rows)
                plsc.subcore_barrier()

            _edge_loop(table_ref, acc_sh, src_hbm, dst_hbm, base0, src_v,
                       dst_v, rows_v, st_v, dt_v, sems, n_win, tail,
                       before_loop=seed_and_sync)
            plsc.subcore_barrier()

        @pl.when(c == 0)
        def _():
            run(tables_hbm.at[0])

        @pl.when(c != 0)
        def _():
            run(tables_hbm.at[1])

        _dump_acc(acc_sh, out_hbm.at[c], s, n, rows_per_sub, tail_rows)

    return k(tables, src, dst, zblk)


def _tc_mlp1(p, x, w1, b1, w2, b2, block_rows=5000):
    n = p.shape[1]
    grid = n // block_rows
    h_out = w2.shape[1]
    hh = h_out // 2

    def body(p_ref, x_ref, w1_ref, b1_ref, w2_ref, b2_ref, out_ref):
        h = x_ref[...] + (p_ref[0] + p_ref[1])
        t = jnp.maximum(_dot(h, w1_ref[...]) + b1_ref[...], 0.0)
        h1 = jnp.maximum(_dot(t, w2_ref[...]) + b2_ref[...], 0.0)
        out_ref[0] = h1[:, :hh]
        out_ref[1] = h1[:, hh:]

    return pl.pallas_call(
        body,
        grid=(grid,),
        in_specs=[
            pl.BlockSpec((2, block_rows, p.shape[2]), lambda i: (0, i, 0)),
            pl.BlockSpec((block_rows, x.shape[1]), lambda i: (i, 0)),
            pl.BlockSpec(w1.shape, lambda i: (0, 0)),
            pl.BlockSpec((1, h_out), lambda i: (0, 0)),
            pl.BlockSpec(w2.shape, lambda i: (0, 0)),
            pl.BlockSpec((1, h_out), lambda i: (0, 0)),
        ],
        out_specs=pl.BlockSpec((2, block_rows, hh), lambda i: (0, i, 0)),
        out_shape=jax.ShapeDtypeStruct((2, n, hh), jnp.float32),
    )(p, x, w1, b1.reshape(1, -1), w2, b2.reshape(1, -1))


def _tc_mlp2_query_head(p2, h1h, w3, b3, w4, b4, qx, q_edges, qw, lw,
                        block_rows=5000):
    n = p2.shape[1]
    grid = n // block_rows
    h = w3.shape[0]
    d_out = w4.shape[1]
    n_q = qx.shape[0]
    e_q = q_edges.shape[1]

    def body(p2_ref, h1_ref, w3_ref, b3_ref, w4_ref, b4_ref, qx_ref, ei_ref,
             qw1, qb1, qw2, qb2, qw3, qb3, qw4, qb4,
             lw1, lb1, lw2, lb2, lw3, lb3, out_ref, acc_ref):
        i = pl.program_id(0)
        hf = jnp.concatenate([h1_ref[0] + p2_ref[0], h1_ref[1] + p2_ref[1]],
                             axis=1)
        t = jnp.maximum(_dot(hf, w3_ref[...]) + b3_ref[...], 0.0)
        dx = _dot(t, w4_ref[...]) + b4_ref[...]
        part = jnp.sum(dx, axis=0, keepdims=True)

        @pl.when(i == 0)
        def _():
            acc_ref[...] = part

        @pl.when(i != 0)
        def _():
            acc_ref[...] += part

        @pl.when(i == grid - 1)
        def _():
            src = ei_ref[0, :]
            dst = ei_ref[1, :]
            iota_en = jax.lax.broadcasted_iota(jnp.int32, (e_q, n_q), 1)
            s_oh = (iota_en == src[:, None]).astype(jnp.float32)
            d_oh = (iota_en == dst[:, None]).astype(jnp.float32).T

            def gin_layer(x, w1, b1, w2, b2):
                hi = jax.lax.Precision.HIGHEST
                agg = _dot(d_oh, _dot(s_oh, x, hi), hi)
                tq = jnp.maximum(_dot(x + agg, w1[...]) + b1[...], 0.0)
                return _dot(tq, w2[...]) + b2[...]

            x = qx_ref[...]
            hq = jnp.maximum(gin_layer(x, qw1, qb1, qw2, qb2), 0.0)
            qx2 = gin_layer(hq, qw3, qb3, qw4, qb4)
            qp = jnp.sum(qx2, axis=0, keepdims=True)

            cat = jnp.concatenate([qp, acc_ref[...]], axis=1)
            o = jnp.maximum(_dot(cat, lw1[...]) + lb1[...], 0.0)
            o = jnp.maximum(_dot(o, lw2[...]) + lb2[...], 0.0)
            o = jnp.maximum(_dot(o, lw3[...]) + lb3[...], 0.0)
            out_ref[...] = o

    qw1, qb1, qw2, qb2, qw3, qb3, qw4, qb4 = qw
    lw1, lb1, lw2, lb2, lw3, lb3 = lw
    consts = [qx, q_edges,
              qw1, qb1.reshape(1, -1), qw2, qb2.reshape(1, -1),
              qw3, qb3.reshape(1, -1), qw4, qb4.reshape(1, -1),
              lw1, lb1.reshape(1, -1), lw2, lb2.reshape(1, -1),
              lw3, lb3.reshape(1, -1)]

    def const_spec(a):
        nd = a.ndim
        return pl.BlockSpec(a.shape, (lambda *_, _nd=nd: (0,) * _nd))

    return pl.pallas_call(
        body,
        grid=(grid,),
        in_specs=[
            pl.BlockSpec((2, block_rows, h // 2), lambda i: (0, i, 0)),
            pl.BlockSpec((2, block_rows, h // 2), lambda i: (0, i, 0)),
            pl.BlockSpec(w3.shape, lambda i: (0, 0)),
            pl.BlockSpec((1, h), lambda i: (0, 0)),
            pl.BlockSpec(w4.shape, lambda i: (0, 0)),
            pl.BlockSpec((1, d_out), lambda i: (0, 0)),
        ] + [const_spec(a) for a in consts],
        out_specs=pl.BlockSpec((1, 8), lambda i: (0, 0)),
        out_shape=jax.ShapeDtypeStruct((1, 8), jnp.float32),
        scratch_shapes=[pltpu.VMEM((1, d_out), jnp.float32)],
    )(p2, h1h, w3, b3.reshape(1, -1), w4, b4.reshape(1, -1), *consts)


def kernel(query_in_feat, data_in_feat, query_edge_list, data_edge_list,
           qW1, qb1, qW2, qb2, qW3, qb3, qW4, qb4,
           dW1, db1, dW2, db2, dW3, db3, dW4, db4,
           lW1, lb1, lW2, lb2, lW3, lb3):
    src = data_edge_list[0]
    dst = data_edge_list[1]

    p1 = _sc_seed_plus_segment_sum(data_in_feat, src, dst)
    h1h = _tc_mlp1(p1, data_in_feat, dW1, db1, dW2, db2)

    p2 = _sc_seed_plus_segment_sum_halves(h1h, src, dst)
    return _tc_mlp2_query_head(
        p2, h1h, dW3, db3, dW4, db4,
        query_in_feat, query_edge_list,
        (qW1, qb1, qW2, qb2, qW3, qb3, qW4, qb4),
        (lW1, lb1, lW2, lb2, lW3, lb3))


# speedup vs baseline: 2.6905x; 2.6905x over previous
import functools

import jax
import jax.numpy as jnp
from jax import lax
from jax.experimental import pallas as pl
from jax.experimental.pallas import tpu as pltpu
from jax.experimental.pallas import tpu_sc as plsc

N_CORES = 2
N_SUB = 16
N_WORKERS = N_CORES * N_SUB
EDGE_WIN = 128

def _dot(a, b, precision=None):
    return jax.lax.dot_general(
        a, b, (((1,), (0,)), ((), ())),
        precision=precision, preferred_element_type=jnp.float32)


def _seed_acc(src_ref, acc_sh, s, n, rows_per_sub, tail_rows):
    row0 = s * rows_per_sub
    pltpu.sync_copy(src_ref.at[pl.ds(row0, rows_per_sub)],
                    acc_sh.at[pl.ds(row0, rows_per_sub)])
    if tail_rows:
        @pl.when(s == 0)
        def _():
            pltpu.sync_copy(src_ref.at[pl.ds(n - tail_rows, tail_rows)],
                            acc_sh.at[pl.ds(n - tail_rows, tail_rows)])


def _dump_acc(acc_sh, out_view, s, n, rows_per_sub, tail_rows):
    row0 = s * rows_per_sub
    pltpu.sync_copy(acc_sh.at[pl.ds(row0, rows_per_sub)],
                    out_view.at[pl.ds(row0, rows_per_sub)])
    if tail_rows:
        @pl.when(s == 0)
        def _():
            pltpu.sync_copy(acc_sh.at[pl.ds(n - tail_rows, tail_rows)],
                            out_view.at[pl.ds(n - tail_rows, tail_rows)])


def _edge_loop(table_ref, acc_sh, src_hbm, dst_hbm, ebase0, src_v, dst_v,
               rows_v, st_v, dt_v, sems, n_win, tail, before_loop=None):
    def src_copy(w, b):
        return pltpu.make_async_copy(
            src_hbm.at[pl.ds(ebase0 + w * EDGE_WIN, EDGE_WIN)],
            src_v.at[b], sems.at[0, b])

    def dst_copy(w, b):
        return pltpu.make_async_copy(
            dst_hbm.at[pl.ds(ebase0 + w * EDGE_WIN, EDGE_WIN)],
            dst_v.at[b], sems.at[1, b])

    def gather_copy(w, b):
        return pltpu.make_async_copy(
            table_ref.at[src_v.at[b]], rows_v.at[b], sems.at[2, b])

    def scatter_copy(b):
        return pltpu.make_async_copy(
            rows_v.at[b], acc_sh.at[dst_v.at[b]], sems.at[3, b])

    def scatter_start(b):
        pltpu.async_copy(rows_v.at[b], acc_sh.at[dst_v.at[b]],
                         sems.at[3, b], add=True)

    for j in range(3):
        src_copy(j, j).start()
    for j in range(2):
        dst_copy(j, j).start()
    for j in range(2):
        src_copy(j, j).wait()
        gather_copy(j, j).start()

    if before_loop is not None:
        before_loop()

    @pl.loop(0, n_win // 3)
    def _(i):
        w0 = 3 * i
        for j in range(3):
            w = w0 + j
            b = j
            b2 = (j + 2) % 3
            gather_copy(w, b).wait()

            @pl.when(w + 3 < n_win)
            def _():
                src_copy(w + 3, b).start()

            @pl.when(w >= 1)
            def _():
                scatter_copy(b2).wait()

            @pl.when(w + 2 < n_win)
            def _():
                dst_copy(w + 2, b2).start()
                src_copy(w + 2, b2).wait()
                gather_copy(w + 2, b2).start()

            dst_copy(w, b).wait()
            scatter_start(b)

    scatter_copy((n_win - 1) % 3).wait()

    if tail:
        base = ebase0 + n_win * EDGE_WIN
        st = pltpu.make_async_copy(
            src_hbm.at[pl.ds(base, tail)], st_v, sems.at[0, 0])
        dt = pltpu.make_async_copy(
            dst_hbm.at[pl.ds(base, tail)], dt_v, sems.at[1, 0])
        st.start()
        dt.start()
        st.wait()
        gt = pltpu.make_async_copy(
            table_ref.at[st_v], rows_v.at[0, pl.ds(0, tail)], sems.at[2, 0])
        gt.start()
        gt.wait()
        dt.wait()
        pltpu.sync_copy(rows_v.at[0, pl.ds(0, tail)], acc_sh.at[dt_v],
                        add=True)


def _sc_scratch(n, d, tail):
    return [
        pltpu.VMEM((3, EDGE_WIN), jnp.int32),
        pltpu.VMEM((3, EDGE_WIN), jnp.int32),
        pltpu.VMEM((3, EDGE_WIN, d), jnp.float32),
        pltpu.VMEM((max(tail, 8),), jnp.int32),
        pltpu.VMEM((max(tail, 8),), jnp.int32),
        pltpu.VMEM_SHARED((n, d), jnp.float32),
        pltpu.SemaphoreType.DMA((4, 3)),
    ]


def _sc_seed_plus_segment_sum(table, src, dst):
    n, d = table.shape
    e = src.shape[0]
    per_w = e // N_WORKERS
    n_win = per_w // EDGE_WIN // 3 * 3
    tail = per_w - n_win * EDGE_WIN
    rows_per_sub = (n // N_SUB) // 8 * 8
    tail_rows = n - rows_per_sub * N_SUB
    zeros = jnp.zeros((n, d), jnp.float32)

    mesh = plsc.VectorSubcoreMesh(core_axis_name="c", subcore_axis_name="s")

    @functools.partial(
        pl.kernel,
        out_type=jax.ShapeDtypeStruct((N_CORES, n, d), jnp.float32),
        mesh=mesh,
        scratch_types=_sc_scratch(n, d, tail),
    )
    def k(table_hbm, src_hbm, dst_hbm, zeros_hbm, out_hbm,
          src_v, dst_v, rows_v, st_v, dt_v, acc_sh, sems):
        c = lax.axis_index("c")
        s = lax.axis_index("s")
        wid = c * N_SUB + s
        base0 = wid * per_w

        def seed_and_sync():
            @pl.when(c == 0)
            def _():
                _seed_acc(table_hbm, acc_sh, s, n, rows_per_sub, tail_rows)

            @pl.when(c != 0)
            def _():
                _seed_acc(zeros_hbm, acc_sh, s, n, rows_per_sub, tail_rows)

            plsc.subcore_barrier()

        _edge_loop(table_hbm, acc_sh, src_hbm, dst_hbm, base0, src_v, dst_v,
                   rows_v, st_v, dt_v, sems, n_win, tail,
                   before_loop=seed_and_sync)
        plsc.subcore_barrier()
        _dump_acc(acc_sh, out_hbm.at[c], s, n, rows_per_sub, tail_rows)

    return k(table, src, dst, zeros)


def _sc_seed_plus_segment_sum_halves(tables, src, dst):
    _, n, d = tables.shape
    e = src.shape[0]
    per_s = e // N_SUB
    n_win = per_s // EDGE_WIN // 3 * 3
    tail = per_s - n_win * EDGE_WIN
    rows_per_sub = (n // N_SUB) // 8 * 8
    tail_rows = n - rows_per_sub * N_SUB

    mesh = plsc.VectorSubcoreMesh(core_axis_name="c", subcore_axis_name="s")

    @functools.partial(
        pl.kernel,
        out_type=jax.ShapeDtypeStruct((N_CORES, n, d), jnp.float32),
        mesh=mesh,
        scratch_types=_sc_scratch(n, d, tail),
    )
    def k(tables_hbm, src_hbm, dst_hbm, out_hbm,
          src_v, dst_v, rows_v, st_v, dt_v, acc_sh, sems):
        c = lax.axis_index("c")
        s = lax.axis_index("s")
        base0 = s * per_s

        def run(table_ref):
            def seed_and_sync():
                _seed_acc(table_ref, acc_sh, s, n, rows_per_sub, tail_rows)
                plsc.subcore_barrier()

            _edge_loop(table_ref, acc_sh, src_hbm, dst_hbm, base0, src_v,
                       dst_v, rows_v, st_v, dt_v, sems, n_win, tail,
                       before_loop=seed_and_sync)
            plsc.subcore_barrier()

        @pl.when(c == 0)
        def _():
            run(tables_hbm.at[0])

        @pl.when(c != 0)
        def _():
            run(tables_hbm.at[1])

        _dump_acc(acc_sh, out_hbm.at[c], s, n, rows_per_sub, tail_rows)

    return k(tables, src, dst)


def _tc_mlp1(p, w1, b1, w2, b2, block_rows=1000):
    n = p.shape[1]
    grid = n // block_rows
    h_out = w2.shape[1]
    hh = h_out // 2

    def body(p_ref, w1_ref, b1_ref, w2_ref, b2_ref, out_ref):
        h = p_ref[0] + p_ref[1]
        t = jnp.maximum(_dot(h, w1_ref[...]) + b1_ref[...], 0.0)
        h1 = jnp.maximum(_dot(t, w2_ref[...]) + b2_ref[...], 0.0)
        out_ref[0] = h1[:, :hh]
        out_ref[1] = h1[:, hh:]

    return pl.pallas_call(
        body,
        grid=(grid,),
        in_specs=[
            pl.BlockSpec((2, block_rows, p.shape[2]), lambda i: (0, i, 0)),
            pl.BlockSpec(w1.shape, lambda i: (0, 0)),
            pl.BlockSpec((1, h_out), lambda i: (0, 0)),
            pl.BlockSpec(w2.shape, lambda i: (0, 0)),
            pl.BlockSpec((1, h_out), lambda i: (0, 0)),
        ],
        out_specs=pl.BlockSpec((2, block_rows, hh), lambda i: (0, i, 0)),
        out_shape=jax.ShapeDtypeStruct((2, n, hh), jnp.float32),
    )(p, w1, b1.reshape(1, -1), w2, b2.reshape(1, -1))


def _tc_mlp2_query_head(p2, w3, b3, w4, b4, qx, q_edges, qw, lw,
                        block_rows=1000):
    n = p2.shape[1]
    grid = n // block_rows
    h = w3.shape[0]
    d_out = w4.shape[1]
    n_q = qx.shape[0]
    e_q = q_edges.shape[1]

    def body(p2_ref, w3_ref, b3_ref, w4_ref, b4_ref, qx_ref, ei_ref,
             qw1, qb1, qw2, qb2, qw3, qb3, qw4, qb4,
             lw1, lb1, lw2, lb2, lw3, lb3, out_ref, acc_ref):
        i = pl.program_id(0)
        hf = jnp.concatenate([p2_ref[0], p2_ref[1]], axis=1)
        t = jnp.maximum(_dot(hf, w3_ref[...]) + b3_ref[...], 0.0)
        dx = _dot(t, w4_ref[...]) + b4_ref[...]
        part = jnp.sum(dx, axis=0, keepdims=True)

        @pl.when(i == 0)
        def _():
            acc_ref[...] = part

        @pl.when(i != 0)
        def _():
            acc_ref[...] += part

        @pl.when(i == grid - 1)
        def _():
            src = ei_ref[0, :]
            dst = ei_ref[1, :]
            iota_en = jax.lax.broadcasted_iota(jnp.int32, (e_q, n_q), 1)
            s_oh = (iota_en == src[:, None]).astype(jnp.float32)
            d_oh = (iota_en == dst[:, None]).astype(jnp.float32).T

            def gin_layer(x, w1, b1, w2, b2):
                hi = jax.lax.Precision.HIGHEST
                agg = _dot(d_oh, _dot(s_oh, x, hi), hi)
                tq = jnp.maximum(_dot(x + agg, w1[...]) + b1[...], 0.0)
                return _dot(tq, w2[...]) + b2[...]

            x = qx_ref[...]
            hq = jnp.maximum(gin_layer(x, qw1, qb1, qw2, qb2), 0.0)
            qx2 = gin_layer(hq, qw3, qb3, qw4, qb4)
            qp = jnp.sum(qx2, axis=0, keepdims=True)

            cat = jnp.concatenate([qp, acc_ref[...]], axis=1)
            o = jnp.maximum(_dot(cat, lw1[...]) + lb1[...], 0.0)
            o = jnp.maximum(_dot(o, lw2[...]) + lb2[...], 0.0)
            o = jnp.maximum(_dot(o, lw3[...]) + lb3[...], 0.0)
            out_ref[...] = o

    qw1, qb1, qw2, qb2, qw3, qb3, qw4, qb4 = qw
    lw1, lb1, lw2, lb2, lw3, lb3 = lw
    consts = [qx, q_edges,
              qw1, qb1.reshape(1, -1), qw2, qb2.reshape(1, -1),
              qw3, qb3.reshape(1, -1), qw4, qb4.reshape(1, -1),
              lw1, lb1.reshape(1, -1), lw2, lb2.reshape(1, -1),
              lw3, lb3.reshape(1, -1)]

    def const_spec(a):
        nd = a.ndim
        return pl.BlockSpec(a.shape, (lambda *_, _nd=nd: (0,) * _nd))

    return pl.pallas_call(
        body,
        grid=(grid,),
        in_specs=[
            pl.BlockSpec((2, block_rows, h // 2), lambda i: (0, i, 0)),
            pl.BlockSpec(w3.shape, lambda i: (0, 0)),
            pl.BlockSpec((1, h), lambda i: (0, 0)),
            pl.BlockSpec(w4.shape, lambda i: (0, 0)),
            pl.BlockSpec((1, d_out), lambda i: (0, 0)),
        ] + [const_spec(a) for a in consts],
        out_specs=pl.BlockSpec((1, 8), lambda i: (0, 0)),
        out_shape=jax.ShapeDtypeStruct((1, 8), jnp.float32),
        scratch_shapes=[pltpu.VMEM((1, d_out), jnp.float32)],
    )(p2, w3, b3.reshape(1, -1), w4, b4.reshape(1, -1), *consts)


def kernel(query_in_feat, data_in_feat, query_edge_list, data_edge_list,
           qW1, qb1, qW2, qb2, qW3, qb3, qW4, qb4,
           dW1, db1, dW2, db2, dW3, db3, dW4, db4,
           lW1, lb1, lW2, lb2, lW3, lb3):
    src = data_edge_list[0]
    dst = data_edge_list[1]

    p1 = _sc_seed_plus_segment_sum(data_in_feat, src, dst)
    h1h = _tc_mlp1(p1, dW1, db1, dW2, db2)

    p2 = _sc_seed_plus_segment_sum_halves(h1h, src, dst)
    return _tc_mlp2_query_head(
        p2, dW3, db3, dW4, db4,
        query_in_feat, query_edge_list,
        (qW1, qb1, qW2, qb2, qW3, qb3, qW4, qb4),
        (lW1, lb1, lW2, lb2, lW3, lb3))


# speedup vs baseline: 2.7557x; 1.0243x over previous
import functools

import jax
import jax.numpy as jnp
from jax import lax
from jax.experimental import pallas as pl
from jax.experimental.pallas import tpu as pltpu
from jax.experimental.pallas import tpu_sc as plsc

N_CORES = 2
N_SUB = 16
N_WORKERS = N_CORES * N_SUB
EDGE_WIN = 128

def _dot(a, b, precision=None):
    return jax.lax.dot_general(
        a, b, (((1,), (0,)), ((), ())),
        precision=precision, preferred_element_type=jnp.float32)


def _seed_acc(src_ref, acc_sh, s, n, rows_per_sub, tail_rows):
    row0 = s * rows_per_sub
    pltpu.sync_copy(src_ref.at[pl.ds(row0, rows_per_sub)],
                    acc_sh.at[pl.ds(row0, rows_per_sub)])
    if tail_rows:
        @pl.when(s == 0)
        def _():
            pltpu.sync_copy(src_ref.at[pl.ds(n - tail_rows, tail_rows)],
                            acc_sh.at[pl.ds(n - tail_rows, tail_rows)])


def _dump_acc(acc_sh, out_view, s, n, rows_per_sub, tail_rows):
    row0 = s * rows_per_sub
    pltpu.sync_copy(acc_sh.at[pl.ds(row0, rows_per_sub)],
                    out_view.at[pl.ds(row0, rows_per_sub)])
    if tail_rows:
        @pl.when(s == 0)
        def _():
            pltpu.sync_copy(acc_sh.at[pl.ds(n - tail_rows, tail_rows)],
                            out_view.at[pl.ds(n - tail_rows, tail_rows)])


def _edge_loop(table_ref, acc_sh, src_hbm, dst_hbm, ebase0, src_v, dst_v,
               rows_v, st_v, dt_v, sems, n_win, tail, before_loop=None):
    def src_copy(w, b):
        return pltpu.make_async_copy(
            src_hbm.at[pl.ds(ebase0 + w * EDGE_WIN, EDGE_WIN)],
            src_v.at[b], sems.at[0, b])

    def dst_copy(w, b):
        return pltpu.make_async_copy(
            dst_hbm.at[pl.ds(ebase0 + w * EDGE_WIN, EDGE_WIN)],
            dst_v.at[b], sems.at[1, b])

    def gather_copy(w, b):
        return pltpu.make_async_copy(
            table_ref.at[src_v.at[b]], rows_v.at[b], sems.at[2, b])

    def scatter_copy(b):
        return pltpu.make_async_copy(
            rows_v.at[b], acc_sh.at[dst_v.at[b]], sems.at[3, b])

    def scatter_start(b):
        pltpu.async_copy(rows_v.at[b], acc_sh.at[dst_v.at[b]],
                         sems.at[3, b], add=True)

    for j in range(3):
        src_copy(j, j).start()
    for j in range(2):
        dst_copy(j, j).start()
    for j in range(2):
        src_copy(j, j).wait()
        gather_copy(j, j).start()

    if before_loop is not None:
        before_loop()

    @pl.loop(0, n_win // 3)
    def _(i):
        w0 = 3 * i
        for j in range(3):
            w = w0 + j
            b = j
            b2 = (j + 2) % 3
            gather_copy(w, b).wait()

            @pl.when(w + 3 < n_win)
            def _():
                src_copy(w + 3, b).start()

            @pl.when(w >= 1)
            def _():
                scatter_copy(b2).wait()

            @pl.when(w + 2 < n_win)
            def _():
                dst_copy(w + 2, b2).start()
                src_copy(w + 2, b2).wait()
                gather_copy(w + 2, b2).start()

            dst_copy(w, b).wait()
            scatter_start(b)

    scatter_copy((n_win - 1) % 3).wait()

    if tail:
        base = ebase0 + n_win * EDGE_WIN
        st = pltpu.make_async_copy(
            src_hbm.at[pl.ds(base, tail)], st_v, sems.at[0, 0])
        dt = pltpu.make_async_copy(
            dst_hbm.at[pl.ds(base, tail)], dt_v, sems.at[1, 0])
        st.start()
        dt.start()
        st.wait()
        gt = pltpu.make_async_copy(
            table_ref.at[st_v], rows_v.at[0, pl.ds(0, tail)], sems.at[2, 0])
        gt.start()
        gt.wait()
        dt.wait()
        pltpu.sync_copy(rows_v.at[0, pl.ds(0, tail)], acc_sh.at[dt_v],
                        add=True)


def _sc_scratch(n, d, tail):
    return [
        pltpu.VMEM((3, EDGE_WIN), jnp.int32),
        pltpu.VMEM((3, EDGE_WIN), jnp.int32),
        pltpu.VMEM((3, EDGE_WIN, d), jnp.float32),
        pltpu.VMEM((max(tail, 8),), jnp.int32),
        pltpu.VMEM((max(tail, 8),), jnp.int32),
        pltpu.VMEM_SHARED((n, d), jnp.float32),
        pltpu.SemaphoreType.DMA((4, 3)),
    ]


def _sc_seed_plus_segment_sum(table, src, dst):
    n, d = table.shape
    e = src.shape[0]
    per_w = e // N_WORKERS
    n_win = per_w // EDGE_WIN // 3 * 3
    tail = per_w - n_win * EDGE_WIN
    rows_per_sub = (n // N_SUB) // 8 * 8
    tail_rows = n - rows_per_sub * N_SUB
    zeros = jnp.zeros((n, d), jnp.float32)

    mesh = plsc.VectorSubcoreMesh(core_axis_name="c", subcore_axis_name="s")

    @functools.partial(
        pl.kernel,
        out_type=jax.ShapeDtypeStruct((N_CORES, n, d), jnp.float32),
        mesh=mesh,
        scratch_types=_sc_scratch(n, d, tail),
    )
    def k(table_hbm, src_hbm, dst_hbm, zeros_hbm, out_hbm,
          src_v, dst_v, rows_v, st_v, dt_v, acc_sh, sems):
        c = lax.axis_index("c")
        s = lax.axis_index("s")
        wid = c * N_SUB + s
        base0 = wid * per_w

        def seed_and_sync():
            @pl.when(c == 0)
            def _():
                _seed_acc(table_hbm, acc_sh, s, n, rows_per_sub, tail_rows)

            @pl.when(c != 0)
            def _():
                _seed_acc(zeros_hbm, acc_sh, s, n, rows_per_sub, tail_rows)

            plsc.subcore_barrier()

        _edge_loop(table_hbm, acc_sh, src_hbm, dst_hbm, base0, src_v, dst_v,
                   rows_v, st_v, dt_v, sems, n_win, tail,
                   before_loop=seed_and_sync)
        plsc.subcore_barrier()
        _dump_acc(acc_sh, out_hbm.at[c], s, n, rows_per_sub, tail_rows)

    return k(table, src, dst, zeros)


def _sc_seed_plus_segment_sum_halves(tables, src, dst):
    _, n, d = tables.shape
    e = src.shape[0]
    per_s = e // N_SUB
    n_win = per_s // EDGE_WIN // 3 * 3
    tail = per_s - n_win * EDGE_WIN
    rows_per_sub = (n // N_SUB) // 8 * 8
    tail_rows = n - rows_per_sub * N_SUB

    mesh = plsc.VectorSubcoreMesh(core_axis_name="c", subcore_axis_name="s")

    @functools.partial(
        pl.kernel,
        out_type=jax.ShapeDtypeStruct((N_CORES, n, d), jnp.float32),
        mesh=mesh,
        scratch_types=_sc_scratch(n, d, tail),
    )
    def k(tables_hbm, src_hbm, dst_hbm, out_hbm,
          src_v, dst_v, rows_v, st_v, dt_v, acc_sh, sems):
        c = lax.axis_index("c")
        s = lax.axis_index("s")
        base0 = s * per_s

        def run(table_ref):
            def seed_and_sync():
                _seed_acc(table_ref, acc_sh, s, n, rows_per_sub, tail_rows)
                plsc.subcore_barrier()

            _edge_loop(table_ref, acc_sh, src_hbm, dst_hbm, base0, src_v,
                       dst_v, rows_v, st_v, dt_v, sems, n_win, tail,
                       before_loop=seed_and_sync)
            plsc.subcore_barrier()

        @pl.when(c == 0)
        def _():
            run(tables_hbm.at[0])

        @pl.when(c != 0)
        def _():
            run(tables_hbm.at[1])

        _dump_acc(acc_sh, out_hbm.at[c], s, n, rows_per_sub, tail_rows)

    return k(tables, src, dst)


def _tc_mlp1(p, w1, b1, w2, b2, block_rows=5000):
    n = p.shape[1]
    grid = n // block_rows
    h_out = w2.shape[1]
    hh = h_out // 2

    def body(p_ref, w1_ref, b1_ref, w2_ref, b2_ref, out_ref):
        h = p_ref[0] + p_ref[1]
        t = jnp.maximum(_dot(h, w1_ref[...]) + b1_ref[...], 0.0)
        h1 = jnp.maximum(_dot(t, w2_ref[...]) + b2_ref[...], 0.0)
        out_ref[0] = h1[:, :hh]
        out_ref[1] = h1[:, hh:]

    return pl.pallas_call(
        body,
        grid=(grid,),
        in_specs=[
            pl.BlockSpec((2, block_rows, p.shape[2]), lambda i: (0, i, 0)),
            pl.BlockSpec(w1.shape, lambda i: (0, 0)),
            pl.BlockSpec((1, h_out), lambda i: (0, 0)),
            pl.BlockSpec(w2.shape, lambda i: (0, 0)),
            pl.BlockSpec((1, h_out), lambda i: (0, 0)),
        ],
        out_specs=pl.BlockSpec((2, block_rows, hh), lambda i: (0, i, 0)),
        out_shape=jax.ShapeDtypeStruct((2, n, hh), jnp.float32),
    )(p, w1, b1.reshape(1, -1), w2, b2.reshape(1, -1))


def _tc_mlp2_query_head(p2, w3, b3, w4, b4, qx, q_edges, qw, lw,
                        block_rows=5000):
    n = p2.shape[1]
    grid = n // block_rows
    h = w3.shape[0]
    d_out = w4.shape[1]
    n_q = qx.shape[0]
    e_q = q_edges.shape[1]

    def body(p2_ref, w3_ref, b3_ref, w4_ref, b4_ref, qx_ref, ei_ref,
             qw1, qb1, qw2, qb2, qw3, qb3, qw4, qb4,
             lw1, lb1, lw2, lb2, lw3, lb3, out_ref, acc_ref):
        i = pl.program_id(0)
        hf = jnp.concatenate([p2_ref[0], p2_ref[1]], axis=1)
        t = jnp.maximum(_dot(hf, w3_ref[...]) + b3_ref[...], 0.0)
        dx = _dot(t, w4_ref[...]) + b4_ref[...]
        part = jnp.sum(dx, axis=0, keepdims=True)

        @pl.when(i == 0)
        def _():
            acc_ref[...] = part

        @pl.when(i != 0)
        def _():
            acc_ref[...] += part

        @pl.when(i == grid - 1)
        def _():
            src = ei_ref[0, :]
            dst = ei_ref[1, :]
            iota_en = jax.lax.broadcasted_iota(jnp.int32, (e_q, n_q), 1)
            s_oh = (iota_en == src[:, None]).astype(jnp.float32)
            d_oh = (iota_en == dst[:, None]).astype(jnp.float32).T

            def gin_layer(x, w1, b1, w2, b2):
                hi = jax.lax.Precision.HIGHEST
                agg = _dot(d_oh, _dot(s_oh, x, hi), hi)
                tq = jnp.maximum(_dot(x + agg, w1[...]) + b1[...], 0.0)
                return _dot(tq, w2[...]) + b2[...]

            x = qx_ref[...]
            hq = jnp.maximum(gin_layer(x, qw1, qb1, qw2, qb2), 0.0)
            qx2 = gin_layer(hq, qw3, qb3, qw4, qb4)
            qp = jnp.sum(qx2, axis=0, keepdims=True)

            cat = jnp.concatenate([qp, acc_ref[...]], axis=1)
            o = jnp.maximum(_dot(cat, lw1[...]) + lb1[...], 0.0)
            o = jnp.maximum(_dot(o, lw2[...]) + lb2[...], 0.0)
            o = jnp.maximum(_dot(o, lw3[...]) + lb3[...], 0.0)
            out_ref[...] = o

    qw1, qb1, qw2, qb2, qw3, qb3, qw4, qb4 = qw
    lw1, lb1, lw2, lb2, lw3, lb3 = lw
    consts = [qx, q_edges,
              qw1, qb1.reshape(1, -1), qw2, qb2.reshape(1, -1),
              qw3, qb3.reshape(1, -1), qw4, qb4.reshape(1, -1),
              lw1, lb1.reshape(1, -1), lw2, lb2.reshape(1, -1),
              lw3, lb3.reshape(1, -1)]

    def const_spec(a):
        nd = a.ndim
        return pl.BlockSpec(a.shape, (lambda *_, _nd=nd: (0,) * _nd))

    return pl.pallas_call(
        body,
        grid=(grid,),
        in_specs=[
            pl.BlockSpec((2, block_rows, h // 2), lambda i: (0, i, 0)),
            pl.BlockSpec(w3.shape, lambda i: (0, 0)),
            pl.BlockSpec((1, h), lambda i: (0, 0)),
            pl.BlockSpec(w4.shape, lambda i: (0, 0)),
            pl.BlockSpec((1, d_out), lambda i: (0, 0)),
        ] + [const_spec(a) for a in consts],
        out_specs=pl.BlockSpec((1, 8), lambda i: (0, 0)),
        out_shape=jax.ShapeDtypeStruct((1, 8), jnp.float32),
        scratch_shapes=[pltpu.VMEM((1, d_out), jnp.float32)],
    )(p2, w3, b3.reshape(1, -1), w4, b4.reshape(1, -1), *consts)


def kernel(query_in_feat, data_in_feat, query_edge_list, data_edge_list,
           qW1, qb1, qW2, qb2, qW3, qb3, qW4, qb4,
           dW1, db1, dW2, db2, dW3, db3, dW4, db4,
           lW1, lb1, lW2, lb2, lW3, lb3):
    src = data_edge_list[0]
    dst = data_edge_list[1]

    p1 = _sc_seed_plus_segment_sum(data_in_feat, src, dst)
    h1h = _tc_mlp1(p1, dW1, db1, dW2, db2)

    p2 = _sc_seed_plus_segment_sum_halves(h1h, src, dst)
    return _tc_mlp2_query_head(
        p2, dW3, db3, dW4, db4,
        query_in_feat, query_edge_list,
        (qW1, qb1, qW2, qb2, qW3, qb3, qW4, qb4),
        (lW1, lb1, lW2, lb2, lW3, lb3))
